# Initial kernel scaffold; baseline (speedup 1.0000x reference)
import jax
import jax.numpy as jnp
from jax.experimental import pallas as pl


def kernel(g, node_features, offsets, W1, b1, W2, b2):
    raise NotImplementedError("write your pallas kernel here")


# speedup vs baseline: 6.6344x; 6.6344x over previous
import jax
import jax.numpy as jnp
from jax import lax
from jax.experimental import pallas as pl
from jax.experimental.pallas import tpu as pltpu


def _fused_body(starts_ref, ends_ref, x_ref, w1_ref, b1_ref, w2_ref, b2_ref,
                out_ref, m_ref, s_ref):
    i = pl.program_id(0)
    nt = pl.num_programs(0)
    neg_inf = jnp.float32(-jnp.inf)

    @pl.when(i == 0)
    def _init():
        m_ref[...] = jnp.full_like(m_ref, neg_inf)
        s_ref[...] = jnp.zeros_like(s_ref)
        out_ref[...] = jnp.zeros_like(out_ref)

    x = x_ref[...]
    h = jnp.tanh(jnp.dot(x, w1_ref[...],
                         preferred_element_type=jnp.float32) + b1_ref[...])
    sc = lax.dot_general(w2_ref[...], h, (((0,), (1,)), ((), ())),
                         preferred_element_type=jnp.float32) + b2_ref[0, 0]

    r = x.shape[0]
    b = out_ref.shape[0]
    ridx = i * r + lax.broadcasted_iota(jnp.int32, (b, r), 1)
    mask = (ridx >= starts_ref[...]) & (ridx < ends_ref[...])

    masked_sc = jnp.where(mask, sc, neg_inf)
    tile_max = jnp.max(masked_sc, axis=1, keepdims=True)
    m_old = m_ref[...]
    m_new = jnp.maximum(m_old, tile_max)
    scale = jnp.where(m_old == neg_inf, 0.0, jnp.exp(m_old - m_new))
    e = jnp.where(mask, jnp.exp(sc - m_new), 0.0)
    s_ref[...] = s_ref[...] * scale + jnp.sum(e, axis=1, keepdims=True)
    out_ref[...] = out_ref[...] * scale + jnp.dot(
        e, x, preferred_element_type=jnp.float32)
    m_ref[...] = m_new

    @pl.when(i == nt - 1)
    def _fin():
        s = s_ref[...]
        out_ref[...] = jnp.where(s > 0, out_ref[...] / s, 0.0)


def kernel(g, node_features, offsets, W1, b1, W2, b2):
    n, h = node_features.shape
    b = offsets.shape[0]
    off = offsets.astype(jnp.int32)
    starts = jnp.concatenate([off[:1], off[:-1]]).reshape(b, 1)
    ends = jnp.concatenate([off[1:], jnp.full((1,), n, jnp.int32)]).reshape(b, 1)

    tile_r = 2048
    grid = (n // tile_r,)

    out = pl.pallas_call(
        _fused_body,
        grid=grid,
        in_specs=[
            pl.BlockSpec((b, 1), lambda i: (0, 0)),
            pl.BlockSpec((b, 1), lambda i: (0, 0)),
            pl.BlockSpec((tile_r, h), lambda i: (i, 0)),
            pl.BlockSpec((h, h), lambda i: (0, 0)),
            pl.BlockSpec((1, h), lambda i: (0, 0)),
            pl.BlockSpec((h, 1), lambda i: (0, 0)),
            pl.BlockSpec((1, 1), lambda i: (0, 0)),
        ],
        out_specs=pl.BlockSpec((b, h), lambda i: (0, 0)),
        out_shape=jax.ShapeDtypeStruct((b, h), jnp.float32),
        scratch_shapes=[
            pltpu.VMEM((b, 1), jnp.float32),
            pltpu.VMEM((b, 1), jnp.float32),
        ],
    )(starts, ends, node_features, W1, b1.reshape(1, h), W2,
      b2.reshape(1, 1))
    return out


# speedup vs baseline: 8.6951x; 1.3106x over previous
import jax
import jax.numpy as jnp
from jax import lax
from jax.experimental import pallas as pl
from jax.experimental.pallas import tpu as pltpu


def _fused_body(starts_ref, ends_ref, x_ref, w1_ref, b1_ref, w2_ref, b2_ref,
                out_ref, m_ref, s_ref):
    i = pl.program_id(0)
    nt = pl.num_programs(0)
    neg_inf = jnp.float32(-jnp.inf)

    @pl.when(i == 0)
    def _init():
        m_ref[...] = jnp.full_like(m_ref, neg_inf)
        s_ref[...] = jnp.zeros_like(s_ref)
        out_ref[...] = jnp.zeros_like(out_ref)

    x = x_ref[...]
    h = jnp.tanh(jnp.dot(x.astype(jnp.bfloat16), w1_ref[...].astype(jnp.bfloat16),
                         preferred_element_type=jnp.float32) + b1_ref[...])
    sc = lax.dot_general(w2_ref[...], h, (((0,), (1,)), ((), ())),
                         preferred_element_type=jnp.float32) + b2_ref[0, 0]

    r = x.shape[0]
    b = out_ref.shape[0]
    ridx = i * r + lax.broadcasted_iota(jnp.int32, (b, r), 1)
    mask = (ridx >= starts_ref[...]) & (ridx < ends_ref[...])

    masked_sc = jnp.where(mask, sc, neg_inf)
    tile_max = jnp.max(masked_sc, axis=1, keepdims=True)
    m_old = m_ref[...]
    m_new = jnp.maximum(m_old, tile_max)
    scale = jnp.where(m_old == neg_inf, 0.0, jnp.exp(m_old - m_new))
    e = jnp.where(mask, jnp.exp(sc - m_new), 0.0)
    s_ref[...] = s_ref[...] * scale + jnp.sum(e, axis=1, keepdims=True)
    out_ref[...] = out_ref[...] * scale + jnp.dot(
        e, x, preferred_element_type=jnp.float32)
    m_ref[...] = m_new

    @pl.when(i == nt - 1)
    def _fin():
        s = s_ref[...]
        out_ref[...] = jnp.where(s > 0, out_ref[...] / s, 0.0)


def kernel(g, node_features, offsets, W1, b1, W2, b2):
    n, h = node_features.shape
    b = offsets.shape[0]
    off = offsets.astype(jnp.int32)
    starts = jnp.concatenate([off[:1], off[:-1]]).reshape(b, 1)
    ends = jnp.concatenate([off[1:], jnp.full((1,), n, jnp.int32)]).reshape(b, 1)

    tile_r = 4096
    grid = (n // tile_r,)

    out = pl.pallas_call(
        _fused_body,
        grid=grid,
        in_specs=[
            pl.BlockSpec((b, 1), lambda i: (0, 0)),
            pl.BlockSpec((b, 1), lambda i: (0, 0)),
            pl.BlockSpec((tile_r, h), lambda i: (i, 0)),
            pl.BlockSpec((h, h), lambda i: (0, 0)),
            pl.BlockSpec((1, h), lambda i: (0, 0)),
            pl.BlockSpec((h, 1), lambda i: (0, 0)),
            pl.BlockSpec((1, 1), lambda i: (0, 0)),
        ],
        out_specs=pl.BlockSpec((b, h), lambda i: (0, 0)),
        out_shape=jax.ShapeDtypeStruct((b, h), jnp.float32),
        scratch_shapes=[
            pltpu.VMEM((b, 1), jnp.float32),
            pltpu.VMEM((b, 1), jnp.float32),
        ],
    )(starts, ends, node_features, W1, b1.reshape(1, h), W2,
      b2.reshape(1, 1))
    return out
